# baseline (speedup 1.0000x reference)
import jax
import jax.numpy as jnp
from jax import lax
from jax.experimental import pallas as pl
from jax.experimental.pallas import tpu as pltpu
from jax.experimental.pallas import tpu_sc as plsc

_NC = 2
_NS = 16
_NW = _NC * _NS

_VOCAB = 1000000
_EMBED = 16
_BATCH = 16384
_FIELDS = 26

_BC = _BATCH // 128
_BC_W = _BC // _NW
_STRIPE = _BC_W * 128
_TASKS = _FIELDS
_LOOPS = -(-_TASKS // 4)


def _gather_body(idx_hbm, w_hbm, scale_hbm, out_hbm, idx_v, scale_v,
                 r0, r1, r2, r3, o0, o1, o2, o3, sems):
    rows = (r0, r1, r2, r3)
    outs = (o0, o1, o2, o3)
    sem_g = (sems[0], sems[1], sems[2], sems[3])
    sem_w = (sems[4], sems[5], sems[6], sems[7])
    wid = lax.axis_index("s") * _NC + lax.axis_index("c")
    pltpu.sync_copy(idx_hbm.at[:, pl.ds(wid * _BC_W, _BC_W)], idx_v)
    pltpu.sync_copy(scale_hbm, scale_v)
    s = scale_v[...]
    lanes = lax.iota(jnp.int32, 16)

    def fire_gather(f, buf):
        for c in range(_BC_W):
            pltpu.async_copy(
                w_hbm.at[idx_v.at[f, c]],
                rows[buf].at[pl.ds(c * 128, 128)],
                sem_g[buf],
            )

    def wait_gather(buf):
        for c in range(_BC_W):
            pltpu.make_async_copy(
                w_hbm.at[idx_v.at[0, 0]],
                rows[buf].at[pl.ds(c * 128, 128)],
                sem_g[buf],
            ).wait()

    def fire_write(f, buf):
        pltpu.async_copy(
            outs[buf],
            out_hbm.at[pl.ds(f * _EMBED, _EMBED),
                       pl.ds(wid * _STRIPE, _STRIPE)],
            sem_w[buf],
        )

    def wait_write(buf):
        pltpu.make_async_copy(
            outs[buf],
            out_hbm.at[pl.ds(0, _EMBED), pl.ds(0, _STRIPE)],
            sem_w[buf],
        ).wait()

    fire_gather(0, 0)
    fire_gather(1, 1)

    def step(g, carry):
        for b in range(4):
            task = 4 * g + b

            @pl.when(task < _TASKS)
            def _():
                wait_gather(b)

                @pl.when(task + 2 < _TASKS)
                def _():
                    fire_gather(task + 2, (b + 2) % 4)

                @pl.when(task >= 4)
                def _():
                    wait_write(b)

                for e in range(_EMBED):
                    for jp in range(_STRIPE // 16):
                        col = plsc.load_gather(
                            rows[b],
                            [jp * 16 + lanes, jnp.full((16,), e, jnp.int32)],
                        )
                        outs[b][e, pl.ds(jp * 16, 16)] = col * s

                fire_write(task, b)
        return carry

    lax.fori_loop(0, _LOOPS, step, None)
    for b in range(4):
        wait_write(b)


_SC_PARAMS = pltpu.CompilerParams(
    use_tc_tiling_on_sc=False, needs_layout_passes=False
)


@jax.jit
def _run(idx3, weight, scale_vec):
    mesh = plsc.VectorSubcoreMesh(core_axis_name="c", subcore_axis_name="s")
    out = pl.kernel(
        _gather_body,
        out_type=jax.ShapeDtypeStruct((_FIELDS * _EMBED, _BATCH), jnp.float32),
        mesh=mesh,
        scratch_types=[
            pltpu.VMEM((_FIELDS, _BC_W, 128), jnp.int32),
            pltpu.VMEM((_EMBED,), jnp.float32),
            pltpu.VMEM((_STRIPE, _EMBED), jnp.float32),
            pltpu.VMEM((_STRIPE, _EMBED), jnp.float32),
            pltpu.VMEM((_STRIPE, _EMBED), jnp.float32),
            pltpu.VMEM((_STRIPE, _EMBED), jnp.float32),
            pltpu.VMEM((_EMBED, _STRIPE), jnp.float32),
            pltpu.VMEM((_EMBED, _STRIPE), jnp.float32),
            pltpu.VMEM((_EMBED, _STRIPE), jnp.float32),
            pltpu.VMEM((_EMBED, _STRIPE), jnp.float32),
            [pltpu.SemaphoreType.DMA] * 8,
        ],
        compiler_params=_SC_PARAMS,
    )(idx3, weight, scale_vec)
    return out


def kernel(input, weight, weight_scale):
    idx3 = input.T.astype(jnp.int32).reshape(_FIELDS, _BC, 128)
    scale_vec = jnp.broadcast_to(
        weight_scale.astype(jnp.float32), (_EMBED,)
    )
    out = _run(idx3, weight, scale_vec)
    return out.reshape(_FIELDS, _EMBED, _BATCH).transpose(2, 0, 1)


# speedup vs baseline: 1.1865x; 1.1865x over previous
import jax
import jax.numpy as jnp
from jax import lax
from jax.experimental import pallas as pl
from jax.experimental.pallas import tpu as pltpu
from jax.experimental.pallas import tpu_sc as plsc

_NC = 2
_NS = 16
_NW = _NC * _NS

_VOCAB = 1000000
_EMBED = 16
_BATCH = 16384
_FIELDS = 26

_BC = _BATCH // 128
_BC_W = _BC // _NW
_STRIPE = _BC_W * 128
_TASKS = _FIELDS
_LOOPS = -(-_TASKS // 4)


def _gather_body(idx_hbm, w_hbm, scale_hbm, out_hbm, idx_v, scale_v,
                 r0, r1, r2, r3, o0, o1, o2, o3, sems):
    rows = (r0, r1, r2, r3)
    outs = (o0, o1, o2, o3)
    sem_g = (sems[0], sems[1], sems[2], sems[3])
    sem_w = (sems[4], sems[5], sems[6], sems[7])
    wid = lax.axis_index("s") * _NC + lax.axis_index("c")
    pltpu.sync_copy(idx_hbm.at[:, pl.ds(wid * _BC_W, _BC_W)], idx_v)
    pltpu.sync_copy(scale_hbm, scale_v)
    s = scale_v[...]
    lanes = lax.iota(jnp.int32, 16)

    def fire_gather(f, buf):
        for c in range(_BC_W):
            pltpu.async_copy(
                w_hbm.at[idx_v.at[f, c]],
                rows[buf].at[pl.ds(c * 128, 128)],
                sem_g[buf],
            )

    def wait_gather(buf):
        for c in range(_BC_W):
            pltpu.make_async_copy(
                w_hbm.at[idx_v.at[0, 0]],
                rows[buf].at[pl.ds(c * 128, 128)],
                sem_g[buf],
            ).wait()

    def fire_write(f, buf):
        pltpu.async_copy(
            outs[buf],
            out_hbm.at[pl.ds(f * _EMBED, _EMBED),
                       pl.ds(wid * _STRIPE, _STRIPE)],
            sem_w[buf],
        )

    def wait_write(buf):
        pltpu.make_async_copy(
            outs[buf],
            out_hbm.at[pl.ds(0, _EMBED), pl.ds(0, _STRIPE)],
            sem_w[buf],
        ).wait()

    fire_gather(0, 0)
    fire_gather(1, 1)

    def step(g, carry):
        for b in range(4):
            task = 4 * g + b

            @pl.when(task < _TASKS)
            def _():
                wait_gather(b)

                @pl.when(task + 2 < _TASKS)
                def _():
                    fire_gather(task + 2, (b + 2) % 4)

                @pl.when(task >= 4)
                def _():
                    wait_write(b)

                @plsc.parallel_loop(0, _STRIPE * _EMBED // 16, unroll=8)
                def _(i):
                    e = i & (_EMBED - 1)
                    jp = i >> 4
                    col = plsc.load_gather(
                        rows[b],
                        [jp * 16 + lanes, jnp.full((16,), e, jnp.int32)],
                    )
                    outs[b][e, pl.ds(jp * 16, 16)] = col * s

                fire_write(task, b)
        return carry

    lax.fori_loop(0, _LOOPS, step, None)
    for b in range(4):
        wait_write(b)


_SC_PARAMS = pltpu.CompilerParams(
    use_tc_tiling_on_sc=False, needs_layout_passes=False
)


@jax.jit
def _run(idx3, weight, scale_vec):
    mesh = plsc.VectorSubcoreMesh(core_axis_name="c", subcore_axis_name="s")
    out = pl.kernel(
        _gather_body,
        out_type=jax.ShapeDtypeStruct((_FIELDS * _EMBED, _BATCH), jnp.float32),
        mesh=mesh,
        scratch_types=[
            pltpu.VMEM((_FIELDS, _BC_W, 128), jnp.int32),
            pltpu.VMEM((_EMBED,), jnp.float32),
            pltpu.VMEM((_STRIPE, _EMBED), jnp.float32),
            pltpu.VMEM((_STRIPE, _EMBED), jnp.float32),
            pltpu.VMEM((_STRIPE, _EMBED), jnp.float32),
            pltpu.VMEM((_STRIPE, _EMBED), jnp.float32),
            pltpu.VMEM((_EMBED, _STRIPE), jnp.float32),
            pltpu.VMEM((_EMBED, _STRIPE), jnp.float32),
            pltpu.VMEM((_EMBED, _STRIPE), jnp.float32),
            pltpu.VMEM((_EMBED, _STRIPE), jnp.float32),
            [pltpu.SemaphoreType.DMA] * 8,
        ],
        compiler_params=_SC_PARAMS,
    )(idx3, weight, scale_vec)
    return out


def kernel(input, weight, weight_scale):
    idx3 = input.T.astype(jnp.int32).reshape(_FIELDS, _BC, 128)
    scale_vec = jnp.broadcast_to(
        weight_scale.astype(jnp.float32), (_EMBED,)
    )
    out = _run(idx3, weight, scale_vec)
    return out.reshape(_FIELDS, _EMBED, _BATCH).transpose(2, 0, 1)


# speedup vs baseline: 1.2485x; 1.0523x over previous
import jax
import jax.numpy as jnp
from jax import lax
from jax.experimental import pallas as pl
from jax.experimental.pallas import tpu as pltpu
from jax.experimental.pallas import tpu_sc as plsc

_NC = 2
_NS = 16
_NW = _NC * _NS

_VOCAB = 1000000
_EMBED = 16
_BATCH = 16384
_FIELDS = 26

_BC = _BATCH // 128
_BC_W = _BC // _NW
_STRIPE = _BC_W * 128
_TASKS = _FIELDS
_LOOPS = -(-_TASKS // 4)


def _gather_body(idx_hbm, w_hbm, scale_hbm, out_hbm, idx_v, scale_v,
                 r0, r1, r2, r3, o0, o1, o2, o3, sems):
    rows = (r0, r1, r2, r3)
    outs = (o0, o1, o2, o3)
    sem_g = (sems[0], sems[1], sems[2], sems[3])
    sem_w = (sems[4], sems[5], sems[6], sems[7])
    wid = lax.axis_index("s") * _NC + lax.axis_index("c")
    pltpu.sync_copy(idx_hbm.at[:, pl.ds(wid * _BC_W, _BC_W)], idx_v)
    pltpu.sync_copy(scale_hbm, scale_v)
    s = scale_v[...]
    lanes = lax.iota(jnp.int32, 16)

    def fire_gather(f, buf):
        for c in range(_BC_W):
            pltpu.async_copy(
                w_hbm.at[idx_v.at[f, c]],
                rows[buf].at[pl.ds(c * 128, 128)],
                sem_g[buf],
            )

    def wait_gather(buf):
        for c in range(_BC_W):
            pltpu.make_async_copy(
                w_hbm.at[idx_v.at[0, 0]],
                rows[buf].at[pl.ds(c * 128, 128)],
                sem_g[buf],
            ).wait()

    def fire_write(f, buf):
        for u in range(2):
            for c in range(_BC_W):
                pltpu.async_copy(
                    outs[buf].at[pl.ds(u * 8, 8), pl.ds(c * 128, 128)],
                    out_hbm.at[2 * f + u, wid * _BC_W + c],
                    sem_w[buf],
                )

    def wait_write(buf):
        for _ in range(2 * _BC_W):
            pltpu.make_async_copy(
                outs[buf].at[pl.ds(0, 8), pl.ds(0, 128)],
                out_hbm.at[0, 0],
                sem_w[buf],
            ).wait()

    fire_gather(0, 0)
    fire_gather(1, 1)

    def step(g, carry):
        for b in range(4):
            task = 4 * g + b

            @pl.when(task < _TASKS)
            def _():
                wait_gather(b)

                @pl.when(task + 2 < _TASKS)
                def _():
                    fire_gather(task + 2, (b + 2) % 4)

                @pl.when(task >= 4)
                def _():
                    wait_write(b)

                @plsc.parallel_loop(0, _STRIPE * _EMBED // 16, unroll=8)
                def _(i):
                    e = i & (_EMBED - 1)
                    jp = i >> 4
                    col = plsc.load_gather(
                        rows[b],
                        [jp * 16 + lanes, jnp.full((16,), e, jnp.int32)],
                    )
                    outs[b][e, pl.ds(jp * 16, 16)] = col * s

                fire_write(task, b)
        return carry

    lax.fori_loop(0, _LOOPS, step, None)
    for b in range(4):
        wait_write(b)


_SC_PARAMS = pltpu.CompilerParams(
    use_tc_tiling_on_sc=False, needs_layout_passes=False
)


@jax.jit
def _run(idx3, weight, scale_vec):
    mesh = plsc.VectorSubcoreMesh(core_axis_name="c", subcore_axis_name="s")
    out = pl.kernel(
        _gather_body,
        out_type=jax.ShapeDtypeStruct(
            (_FIELDS * 2, _BC, 8, 128), jnp.float32
        ),
        mesh=mesh,
        scratch_types=[
            pltpu.VMEM((_FIELDS, _BC_W, 128), jnp.int32),
            pltpu.VMEM((_EMBED,), jnp.float32),
            pltpu.VMEM((_STRIPE, _EMBED), jnp.float32),
            pltpu.VMEM((_STRIPE, _EMBED), jnp.float32),
            pltpu.VMEM((_STRIPE, _EMBED), jnp.float32),
            pltpu.VMEM((_STRIPE, _EMBED), jnp.float32),
            pltpu.VMEM((_EMBED, _STRIPE), jnp.float32),
            pltpu.VMEM((_EMBED, _STRIPE), jnp.float32),
            pltpu.VMEM((_EMBED, _STRIPE), jnp.float32),
            pltpu.VMEM((_EMBED, _STRIPE), jnp.float32),
            [pltpu.SemaphoreType.DMA] * 8,
        ],
        compiler_params=_SC_PARAMS,
    )(idx3, weight, scale_vec)
    return out


def kernel(input, weight, weight_scale):
    idx3 = input.T.astype(jnp.int32).reshape(_FIELDS, _BC, 128)
    scale_vec = jnp.broadcast_to(
        weight_scale.astype(jnp.float32), (_EMBED,)
    )
    out4 = _run(idx3, weight, scale_vec)
    a = out4.reshape(_FIELDS, 2, _BC, 8, 128)
    b = a.transpose(2, 4, 0, 1, 3)
    return b.reshape(_BATCH, _FIELDS, _EMBED)


# speedup vs baseline: 1.2812x; 1.0262x over previous
import jax
import jax.numpy as jnp
from jax import lax
from jax.experimental import pallas as pl
from jax.experimental.pallas import tpu as pltpu
from jax.experimental.pallas import tpu_sc as plsc

_NC = 2
_NS = 16
_NW = _NC * _NS

_VOCAB = 1000000
_EMBED = 16
_BATCH = 16384
_FIELDS = 26

_BC = _BATCH // 128
_BC_W = _BC // _NW
_STRIPE = _BC_W * 128
_TASKS = _FIELDS
_LOOPS = -(-_TASKS // 4)


def _gather_body(idx_hbm, w_hbm, scale_hbm, out_hbm, idx_v, scale_v,
                 r0, r1, r2, r3, o0, o1, o2, o3, sems):
    rows = (r0, r1, r2, r3)
    outs = (o0, o1, o2, o3)
    sem_g = (sems[0], sems[1], sems[2], sems[3])
    sem_w = (sems[4], sems[5], sems[6], sems[7])
    wid = lax.axis_index("s") * _NC + lax.axis_index("c")
    pltpu.sync_copy(idx_hbm.at[:, pl.ds(wid * _BC_W, _BC_W)], idx_v)
    pltpu.sync_copy(scale_hbm, scale_v)
    s = scale_v[...]
    lanes = lax.iota(jnp.int32, 16)

    def fire_gather(f, buf):
        for c in range(_BC_W):
            pltpu.async_copy(
                w_hbm.at[idx_v.at[f, c]],
                rows[buf].at[pl.ds(c * 128, 128)],
                sem_g[buf],
            )

    def wait_gather(buf):
        for c in range(_BC_W):
            pltpu.make_async_copy(
                w_hbm.at[idx_v.at[0, 0]],
                rows[buf].at[pl.ds(c * 128, 128)],
                sem_g[buf],
            ).wait()

    def fire_write(f, buf):
        for u in range(2):
            for c in range(_BC_W):
                pltpu.async_copy(
                    outs[buf].at[pl.ds(u * 8, 8), pl.ds(c * 128, 128)],
                    out_hbm.at[2 * f + u, wid * _BC_W + c],
                    sem_w[buf],
                )

    def wait_write(buf):
        for _ in range(2 * _BC_W):
            pltpu.make_async_copy(
                outs[buf].at[pl.ds(0, 8), pl.ds(0, 128)],
                out_hbm.at[0, 0],
                sem_w[buf],
            ).wait()

    fire_gather(0, 0)
    fire_gather(1, 1)

    def step(g, carry):
        for b in range(4):
            task = 4 * g + b

            @pl.when(task < _TASKS)
            def _():
                wait_gather(b)

                @pl.when(task + 2 < _TASKS)
                def _():
                    fire_gather(task + 2, (b + 2) % 4)

                @pl.when(task >= 4)
                def _():
                    wait_write(b)

                @plsc.parallel_loop(0, _STRIPE * _EMBED // 16, unroll=16)
                def _(i):
                    e = i & (_EMBED - 1)
                    jp = i >> 4
                    col = plsc.load_gather(
                        rows[b],
                        [jp * 16 + lanes, jnp.full((16,), e, jnp.int32)],
                    )
                    outs[b][e, pl.ds(jp * 16, 16)] = col * s

                fire_write(task, b)
        return carry

    lax.fori_loop(0, _LOOPS, step, None)
    for b in range(4):
        wait_write(b)


_SC_PARAMS = pltpu.CompilerParams(
    use_tc_tiling_on_sc=False, needs_layout_passes=False
)


@jax.jit
def _run(idx3, weight, scale_vec):
    mesh = plsc.VectorSubcoreMesh(core_axis_name="c", subcore_axis_name="s")
    out = pl.kernel(
        _gather_body,
        out_type=jax.ShapeDtypeStruct(
            (_FIELDS * 2, _BC, 8, 128), jnp.float32
        ),
        mesh=mesh,
        scratch_types=[
            pltpu.VMEM((_FIELDS, _BC_W, 128), jnp.int32),
            pltpu.VMEM((_EMBED,), jnp.float32),
            pltpu.VMEM((_STRIPE, _EMBED), jnp.float32),
            pltpu.VMEM((_STRIPE, _EMBED), jnp.float32),
            pltpu.VMEM((_STRIPE, _EMBED), jnp.float32),
            pltpu.VMEM((_STRIPE, _EMBED), jnp.float32),
            pltpu.VMEM((_EMBED, _STRIPE), jnp.float32),
            pltpu.VMEM((_EMBED, _STRIPE), jnp.float32),
            pltpu.VMEM((_EMBED, _STRIPE), jnp.float32),
            pltpu.VMEM((_EMBED, _STRIPE), jnp.float32),
            [pltpu.SemaphoreType.DMA] * 8,
        ],
        compiler_params=_SC_PARAMS,
    )(idx3, weight, scale_vec)
    return out


def kernel(input, weight, weight_scale):
    idx3 = input.T.astype(jnp.int32).reshape(_FIELDS, _BC, 128)
    scale_vec = jnp.broadcast_to(
        weight_scale.astype(jnp.float32), (_EMBED,)
    )
    out4 = _run(idx3, weight, scale_vec)
    a = out4.reshape(_FIELDS, 2, _BC, 8, 128)
    b = a.transpose(2, 4, 0, 1, 3)
    return b.reshape(_BATCH, _FIELDS, _EMBED)
